# baseline (speedup 1.0000x reference)
import jax
import jax.numpy as jnp
from jax.experimental import pallas as pl

_N = 4096
_D = 1024
_E = 8
_A = 16
_TILE = 512


def _fused_kernel(obs_ref, wp_ref, ws_ref, ft_ref, eo_ref,
                  actions_ref, values_ref, logp_ref, newopt_ref,
                  mv_ref, mlp_ref, tp_ref):
    obs = obs_ref[...]
    act_all = jax.lax.dot_general(
        obs, wp_ref[...], (((1,), (0,)), ((), ())),
        preferred_element_type=jnp.float32)
    small = jax.lax.dot_general(
        obs, ws_ref[...], (((1,), (0,)), ((), ())),
        preferred_element_type=jnp.float32)

    meta_logits = small[:, 0:_E]
    term_logits = small[:, _E:2 * _E]
    vals_all = small[:, 2 * _E:3 * _E]
    meta_values = small[:, 3 * _E]

    ft = ft_ref[...][:, None]
    eo = eo_ref[...][:, None]

    t = obs.shape[0]
    lane8 = jax.lax.broadcasted_iota(jnp.int32, (t, _E), 1)

    meta_actions = jnp.argmax(meta_logits, axis=-1).astype(jnp.int32)[:, None]
    mmax = jnp.max(meta_logits, axis=-1, keepdims=True)
    meta_log_probs = -jnp.log(jnp.sum(jnp.exp(meta_logits - mmax), axis=-1))

    term_sel = jnp.sum(jnp.where(lane8 == eo, term_logits, 0.0), axis=-1, keepdims=True)
    term_prob = jax.nn.sigmoid(term_sel)
    requires_new = jnp.logical_or(term_prob > 0.5, ft != 0)
    new_opt = jnp.where(requires_new, meta_actions, eo)
    term_prob = jnp.where(ft != 0, 0.0, term_prob)

    values = jnp.sum(jnp.where(lane8 == new_opt, vals_all, 0.0), axis=-1)

    sel16 = act_all[:, 0:_A]
    for e in range(1, _E):
        sel16 = jnp.where(new_opt == e, act_all[:, _A * e:_A * (e + 1)], sel16)
    actions = jnp.argmax(sel16, axis=-1).astype(jnp.int32)
    amax = jnp.max(sel16, axis=-1, keepdims=True)
    log_probs = -jnp.log(jnp.sum(jnp.exp(sel16 - amax), axis=-1))

    actions_ref[...] = actions
    values_ref[...] = values
    logp_ref[...] = log_probs
    newopt_ref[...] = new_opt[:, 0]
    mv_ref[...] = meta_values
    mlp_ref[...] = meta_log_probs
    tp_ref[...] = term_prob[:, 0]


def kernel(observation, first_transition, executing_option, Wm, Wmv, Wt, Wp, Wv):
    n, d = observation.shape
    e = Wm.shape[1]
    a = Wp.shape[2]
    wp2d = Wp.transpose(1, 0, 2).reshape(d, e * a)
    wsmall = jnp.concatenate(
        [Wm, Wt, Wv[..., 0].T, Wmv, jnp.zeros((d, 32 - 3 * e - 1), jnp.float32)],
        axis=1)
    ft = first_transition.astype(jnp.int32)
    eo = executing_option.astype(jnp.int32)

    grid = (n // _TILE,)
    row_spec = pl.BlockSpec((_TILE,), lambda i: (i,))
    out_specs = [row_spec] * 7
    out_shapes = [
        jax.ShapeDtypeStruct((n,), jnp.int32),
        jax.ShapeDtypeStruct((n,), jnp.float32),
        jax.ShapeDtypeStruct((n,), jnp.float32),
        jax.ShapeDtypeStruct((n,), jnp.int32),
        jax.ShapeDtypeStruct((n,), jnp.float32),
        jax.ShapeDtypeStruct((n,), jnp.float32),
        jax.ShapeDtypeStruct((n,), jnp.float32),
    ]
    outs = pl.pallas_call(
        _fused_kernel,
        grid=grid,
        in_specs=[
            pl.BlockSpec((_TILE, d), lambda i: (i, 0)),
            pl.BlockSpec((d, e * a), lambda i: (0, 0)),
            pl.BlockSpec((d, 32), lambda i: (0, 0)),
            row_spec,
            row_spec,
        ],
        out_specs=out_specs,
        out_shape=out_shapes,
    )(observation, wp2d, wsmall, ft, eo)
    return tuple(outs)


# speedup vs baseline: 2.3212x; 2.3212x over previous
import jax
import jax.numpy as jnp
from jax.experimental import pallas as pl

_N = 4096
_D = 1024
_E = 8
_A = 16
_TILE = 512


def _fused_kernel(obs_ref, wp_ref, ws_ref, ft_ref, eo_ref,
                  actions_ref, values_ref, logp_ref, newopt_ref,
                  mv_ref, mlp_ref, tp_ref):
    obs = obs_ref[...]
    act_t = jax.lax.dot_general(
        wp_ref[...], obs, (((1,), (1,)), ((), ())),
        preferred_element_type=jnp.float32)
    small_t = jax.lax.dot_general(
        ws_ref[...], obs, (((1,), (1,)), ((), ())),
        preferred_element_type=jnp.float32)

    t = obs.shape[0]
    meta_logits = small_t[0:_E]
    term_logits = small_t[_E:2 * _E]
    vals_all = small_t[2 * _E:3 * _E]
    meta_values = small_t[3 * _E]

    ft = ft_ref[...]
    eo = eo_ref[...]

    row8 = jax.lax.broadcasted_iota(jnp.int32, (_E, t), 0)

    mmax = jnp.max(meta_logits, axis=0)
    meta_actions = jnp.min(
        jnp.where(meta_logits == mmax[None, :], row8, _E), axis=0)
    meta_log_probs = -jnp.log(
        jnp.sum(jnp.exp(meta_logits - mmax[None, :]), axis=0))

    term_sel = jnp.sum(
        jnp.where(row8 == eo[None, :], term_logits, 0.0), axis=0)
    term_prob = jax.nn.sigmoid(term_sel)
    requires_new = jnp.logical_or(term_prob > 0.5, ft != 0)
    new_opt = jnp.where(requires_new, meta_actions, eo)
    term_prob = jnp.where(ft != 0, 0.0, term_prob)

    values = jnp.sum(
        jnp.where(row8 == new_opt[None, :], vals_all, 0.0), axis=0)

    sel16 = act_t[0:_A]
    for e in range(1, _E):
        sel16 = jnp.where(new_opt[None, :] == e,
                          act_t[_A * e:_A * (e + 1)], sel16)
    row16 = jax.lax.broadcasted_iota(jnp.int32, (_A, t), 0)
    amax = jnp.max(sel16, axis=0)
    actions = jnp.min(jnp.where(sel16 == amax[None, :], row16, _A), axis=0)
    log_probs = -jnp.log(jnp.sum(jnp.exp(sel16 - amax[None, :]), axis=0))

    actions_ref[...] = actions
    values_ref[...] = values
    logp_ref[...] = log_probs
    newopt_ref[...] = new_opt
    mv_ref[...] = meta_values
    mlp_ref[...] = meta_log_probs
    tp_ref[...] = term_prob


def kernel(observation, first_transition, executing_option, Wm, Wmv, Wt, Wp, Wv):
    n, d = observation.shape
    e = Wm.shape[1]
    a = Wp.shape[2]
    wpt = Wp.transpose(0, 2, 1).reshape(e * a, d)
    wst = jnp.concatenate(
        [Wm.T, Wt.T, Wv[..., 0], Wmv.T, jnp.zeros((32 - 3 * e - 1, d), jnp.float32)],
        axis=0)
    ft = first_transition.astype(jnp.int32)
    eo = executing_option.astype(jnp.int32)

    grid = (n // _TILE,)
    row_spec = pl.BlockSpec((_TILE,), lambda i: (i,))
    out_specs = [row_spec] * 7
    out_shapes = [
        jax.ShapeDtypeStruct((n,), jnp.int32),
        jax.ShapeDtypeStruct((n,), jnp.float32),
        jax.ShapeDtypeStruct((n,), jnp.float32),
        jax.ShapeDtypeStruct((n,), jnp.int32),
        jax.ShapeDtypeStruct((n,), jnp.float32),
        jax.ShapeDtypeStruct((n,), jnp.float32),
        jax.ShapeDtypeStruct((n,), jnp.float32),
    ]
    outs = pl.pallas_call(
        _fused_kernel,
        grid=grid,
        in_specs=[
            pl.BlockSpec((_TILE, d), lambda i: (i, 0)),
            pl.BlockSpec((e * a, d), lambda i: (0, 0)),
            pl.BlockSpec((32, d), lambda i: (0, 0)),
            row_spec,
            row_spec,
        ],
        out_specs=out_specs,
        out_shape=out_shapes,
    )(observation, wpt, wst, ft, eo)
    return tuple(outs)


# speedup vs baseline: 2.3234x; 1.0010x over previous
import jax
import jax.numpy as jnp
from jax.experimental import pallas as pl
from jax.experimental.pallas import tpu as pltpu

_N = 4096
_D = 1024
_E = 8
_A = 16
_TILE = 512


def _fused_kernel(obs_ref, wp_ref, ws_ref, ft_ref, eo_ref,
                  actions_ref, values_ref, logp_ref, newopt_ref,
                  mv_ref, mlp_ref, tp_ref):
    obs = obs_ref[...]
    act_t = jax.lax.dot_general(
        wp_ref[...], obs, (((1,), (1,)), ((), ())),
        preferred_element_type=jnp.float32)
    small_t = jax.lax.dot_general(
        ws_ref[...], obs, (((1,), (1,)), ((), ())),
        preferred_element_type=jnp.float32)

    t = obs.shape[0]
    meta_logits = small_t[0:_E]
    term_logits = small_t[_E:2 * _E]
    vals_all = small_t[2 * _E:3 * _E]
    meta_values = small_t[3 * _E]

    ft = ft_ref[...]
    eo = eo_ref[...]

    row8 = jax.lax.broadcasted_iota(jnp.int32, (_E, t), 0)

    mmax = jnp.max(meta_logits, axis=0)
    meta_actions = jnp.min(
        jnp.where(meta_logits == mmax[None, :], row8, _E), axis=0)
    meta_log_probs = -jnp.log(
        jnp.sum(jnp.exp(meta_logits - mmax[None, :]), axis=0))

    term_sel = jnp.sum(
        jnp.where(row8 == eo[None, :], term_logits, 0.0), axis=0)
    term_prob = jax.nn.sigmoid(term_sel)
    requires_new = jnp.logical_or(term_prob > 0.5, ft != 0)
    new_opt = jnp.where(requires_new, meta_actions, eo)
    term_prob = jnp.where(ft != 0, 0.0, term_prob)

    values = jnp.sum(
        jnp.where(row8 == new_opt[None, :], vals_all, 0.0), axis=0)

    sel16 = act_t[0:_A]
    for e in range(1, _E):
        sel16 = jnp.where(new_opt[None, :] == e,
                          act_t[_A * e:_A * (e + 1)], sel16)
    row16 = jax.lax.broadcasted_iota(jnp.int32, (_A, t), 0)
    amax = jnp.max(sel16, axis=0)
    actions = jnp.min(jnp.where(sel16 == amax[None, :], row16, _A), axis=0)
    log_probs = -jnp.log(jnp.sum(jnp.exp(sel16 - amax[None, :]), axis=0))

    actions_ref[...] = actions
    values_ref[...] = values
    logp_ref[...] = log_probs
    newopt_ref[...] = new_opt
    mv_ref[...] = meta_values
    mlp_ref[...] = meta_log_probs
    tp_ref[...] = term_prob


def kernel(observation, first_transition, executing_option, Wm, Wmv, Wt, Wp, Wv):
    n, d = observation.shape
    e = Wm.shape[1]
    a = Wp.shape[2]
    wpt = Wp.transpose(0, 2, 1).reshape(e * a, d)
    wst = jnp.concatenate(
        [Wm.T, Wt.T, Wv[..., 0], Wmv.T, jnp.zeros((32 - 3 * e - 1, d), jnp.float32)],
        axis=0)
    ft = first_transition.astype(jnp.int32)
    eo = executing_option.astype(jnp.int32)

    grid = (n // _TILE,)
    row_spec = pl.BlockSpec((_TILE,), lambda i: (i,))
    out_specs = [row_spec] * 7
    out_shapes = [
        jax.ShapeDtypeStruct((n,), jnp.int32),
        jax.ShapeDtypeStruct((n,), jnp.float32),
        jax.ShapeDtypeStruct((n,), jnp.float32),
        jax.ShapeDtypeStruct((n,), jnp.int32),
        jax.ShapeDtypeStruct((n,), jnp.float32),
        jax.ShapeDtypeStruct((n,), jnp.float32),
        jax.ShapeDtypeStruct((n,), jnp.float32),
    ]
    outs = pl.pallas_call(
        _fused_kernel,
        grid=grid,
        in_specs=[
            pl.BlockSpec((_TILE, d), lambda i: (i, 0)),
            pl.BlockSpec((e * a, d), lambda i: (0, 0)),
            pl.BlockSpec((32, d), lambda i: (0, 0)),
            row_spec,
            row_spec,
        ],
        out_specs=out_specs,
        out_shape=out_shapes,
        compiler_params=pltpu.CompilerParams(
            dimension_semantics=("parallel",)),
    )(observation, wpt, wst, ft, eo)
    return tuple(outs)


# speedup vs baseline: 2.5648x; 1.1039x over previous
import jax
import jax.numpy as jnp
from jax.experimental import pallas as pl
from jax.experimental.pallas import tpu as pltpu

_N = 4096
_D = 1024
_E = 8
_A = 16
_TILE = 1024


def _fused_kernel(obs_ref, wp_ref, ws_ref, ft_ref, eo_ref,
                  actions_ref, values_ref, logp_ref, newopt_ref,
                  mv_ref, mlp_ref, tp_ref):
    obs = obs_ref[...]
    act_t = jax.lax.dot_general(
        wp_ref[...], obs, (((1,), (1,)), ((), ())),
        preferred_element_type=jnp.float32)
    small_t = jax.lax.dot_general(
        ws_ref[...], obs, (((1,), (1,)), ((), ())),
        preferred_element_type=jnp.float32)

    t = obs.shape[0]
    meta_logits = small_t[0:_E]
    term_logits = small_t[_E:2 * _E]
    vals_all = small_t[2 * _E:3 * _E]
    meta_values = small_t[3 * _E]

    ft = ft_ref[...]
    eo = eo_ref[...]

    row8 = jax.lax.broadcasted_iota(jnp.int32, (_E, t), 0)

    mmax = jnp.max(meta_logits, axis=0)
    meta_actions = jnp.min(
        jnp.where(meta_logits == mmax[None, :], row8, _E), axis=0)
    meta_log_probs = -jnp.log(
        jnp.sum(jnp.exp(meta_logits - mmax[None, :]), axis=0))

    term_sel = jnp.sum(
        jnp.where(row8 == eo[None, :], term_logits, 0.0), axis=0)
    term_prob = jax.nn.sigmoid(term_sel)
    requires_new = jnp.logical_or(term_prob > 0.5, ft != 0)
    new_opt = jnp.where(requires_new, meta_actions, eo)
    term_prob = jnp.where(ft != 0, 0.0, term_prob)

    values = jnp.sum(
        jnp.where(row8 == new_opt[None, :], vals_all, 0.0), axis=0)

    sel16 = act_t[0:_A]
    for e in range(1, _E):
        sel16 = jnp.where(new_opt[None, :] == e,
                          act_t[_A * e:_A * (e + 1)], sel16)
    row16 = jax.lax.broadcasted_iota(jnp.int32, (_A, t), 0)
    amax = jnp.max(sel16, axis=0)
    actions = jnp.min(jnp.where(sel16 == amax[None, :], row16, _A), axis=0)
    log_probs = -jnp.log(jnp.sum(jnp.exp(sel16 - amax[None, :]), axis=0))

    actions_ref[...] = actions
    values_ref[...] = values
    logp_ref[...] = log_probs
    newopt_ref[...] = new_opt
    mv_ref[...] = meta_values
    mlp_ref[...] = meta_log_probs
    tp_ref[...] = term_prob


def kernel(observation, first_transition, executing_option, Wm, Wmv, Wt, Wp, Wv):
    n, d = observation.shape
    e = Wm.shape[1]
    a = Wp.shape[2]
    wpt = Wp.transpose(0, 2, 1).reshape(e * a, d)
    wst = jnp.concatenate(
        [Wm.T, Wt.T, Wv[..., 0], Wmv.T, jnp.zeros((32 - 3 * e - 1, d), jnp.float32)],
        axis=0)
    ft = first_transition.astype(jnp.int32)
    eo = executing_option.astype(jnp.int32)

    grid = (n // _TILE,)
    row_spec = pl.BlockSpec((_TILE,), lambda i: (i,))
    out_specs = [row_spec] * 7
    out_shapes = [
        jax.ShapeDtypeStruct((n,), jnp.int32),
        jax.ShapeDtypeStruct((n,), jnp.float32),
        jax.ShapeDtypeStruct((n,), jnp.float32),
        jax.ShapeDtypeStruct((n,), jnp.int32),
        jax.ShapeDtypeStruct((n,), jnp.float32),
        jax.ShapeDtypeStruct((n,), jnp.float32),
        jax.ShapeDtypeStruct((n,), jnp.float32),
    ]
    outs = pl.pallas_call(
        _fused_kernel,
        grid=grid,
        in_specs=[
            pl.BlockSpec((_TILE, d), lambda i: (i, 0)),
            pl.BlockSpec((e * a, d), lambda i: (0, 0)),
            pl.BlockSpec((32, d), lambda i: (0, 0)),
            row_spec,
            row_spec,
        ],
        out_specs=out_specs,
        out_shape=out_shapes,
        compiler_params=pltpu.CompilerParams(
            dimension_semantics=("parallel",)),
    )(observation, wpt, wst, ft, eo)
    return tuple(outs)
